# baseline (speedup 1.0000x reference)
import functools

import jax
import jax.numpy as jnp
from jax import lax
from jax.experimental import pallas as pl
from jax.experimental.pallas import tpu as pltpu
from jax.experimental.pallas import tpu_sc as plsc

N_ITEMS = 1000000
N_CATS = 1000
EMB = 64
CAT_EMB = 16
BATCH = 16384

NC = 2
NS = 16
NW = NC * NS
B_PER_W = BATCH // NW
LANES = 16
NBUF = 8
HALF = B_PER_W // 2


def _sc_gather(item_tblT, item_ids, cat_ids, cat_tblT):
    @functools.partial(
        pl.kernel,
        out_type=(
            jax.ShapeDtypeStruct((BATCH, EMB), jnp.float32),
            jax.ShapeDtypeStruct((CAT_EMB, BATCH), jnp.float32),
        ),
        mesh=plsc.VectorSubcoreMesh(core_axis_name="c", subcore_axis_name="s"),
        scratch_types=[
            pltpu.VMEM((B_PER_W,), jnp.int32),
            pltpu.VMEM((B_PER_W,), jnp.int32),
            pltpu.VMEM((NBUF, EMB, 128), jnp.float32),
            pltpu.VMEM((HALF, EMB), jnp.float32),
            pltpu.VMEM((CAT_EMB, N_CATS), jnp.float32),
            pltpu.VMEM((CAT_EMB, B_PER_W), jnp.float32),
            pltpu.VMEM((EMB,), jnp.int32),
        ] + [pltpu.SemaphoreType.DMA] * NBUF,
        compiler_params=pltpu.CompilerParams(needs_layout_passes=False),
    )
    def k(tblT_hbm, ids_hbm, cids_hbm, ctblT_hbm, ie_hbm, ceT_hbm,
          idx_v, cidx_v, stg_v, rows_v, cat_v, cdst_v, f64_v, *sems):
        wid = lax.axis_index("s") * NC + lax.axis_index("c")
        base = wid * B_PER_W
        pltpu.sync_copy(ids_hbm.at[pl.ds(base, B_PER_W)], idx_v)
        iota = lax.iota(jnp.int32, LANES)
        for q in range(EMB // LANES):
            f64_v[pl.ds(q * LANES, LANES)] = iota + q * LANES

        def scalar_id(j):
            chunk = idx_v[pl.ds((j >> 4) << 4, LANES)]
            sel = jnp.where(iota == (j & 15), chunk, 0)
            return jnp.sum(sel)

        def fire(j, slot):
            tc = scalar_id(j) >> 7
            col = pl.multiple_of(tc * 128, 128)
            pltpu.async_copy(
                tblT_hbm.at[:, pl.ds(col, 128)].at[f64_v],
                stg_v.at[slot], sems[slot]
            )

        def extract(j, slot):
            pltpu.make_async_copy(
                tblT_hbm.at[:, pl.ds(0, 128)], stg_v.at[slot], sems[slot]
            ).wait()
            lane = jnp.full((LANES,), scalar_id(j) & 127, dtype=jnp.int32)
            for q in range(EMB // LANES):
                vals = plsc.load_gather(
                    stg_v.at[slot], [iota + (q * LANES), lane]
                )
                rows_v[j & (HALF - 1), pl.ds(q * LANES, LANES)] = vals

        for b in range(NBUF):
            fire(b, b)

        pltpu.sync_copy(cids_hbm.at[pl.ds(base, B_PER_W)], cidx_v)
        pltpu.sync_copy(ctblT_hbm, cat_v)

        @pl.loop(0, CAT_EMB)
        def _(f):
            fvec = jnp.full((LANES,), f, dtype=jnp.int32)

            @pl.loop(0, B_PER_W // LANES)
            def _(c):
                cvec = cidx_v[pl.ds(c * LANES, LANES)]
                vals = plsc.load_gather(cat_v, [fvec, cvec])
                cdst_v[f, pl.ds(c * LANES, LANES)] = vals

        pltpu.sync_copy(cdst_v, ceT_hbm.at[:, pl.ds(base, B_PER_W)])

        @pl.loop(0, B_PER_W // NBUF - 1)
        def _(g):
            for b in range(NBUF):
                j = g * NBUF + b
                extract(j, b)
                fire(j + NBUF, b)

            @pl.when(g == HALF // NBUF - 1)
            def _():
                pltpu.sync_copy(rows_v, ie_hbm.at[pl.ds(base, HALF)])

        for b in range(NBUF):
            extract(B_PER_W - NBUF + b, b)

        pltpu.sync_copy(rows_v, ie_hbm.at[pl.ds(base + HALF, HALF)])

    return k(item_tblT, item_ids, cat_ids, cat_tblT)


def _mlp_body(ie_ref, ceT_ref, w1a_ref, w1b_ref, b1_ref, w2_ref, b2_ref,
              out_ref):
    ieT = lax.dot_general(
        w1a_ref[...], ie_ref[...],
        dimension_numbers=(((1,), (1,)), ((), ())),
        preferred_element_type=jnp.float32,
    )
    h = ieT + jnp.dot(w1b_ref[...], ceT_ref[...],
                      preferred_element_type=jnp.float32)
    h = jnp.maximum(h + b1_ref[...], 0.0)
    out_ref[...] = (
        jnp.dot(w2_ref[...], h, preferred_element_type=jnp.float32)
        + b2_ref[...]
    )


def _tc_mlp(ie, ceT, W1a, W1b, b1c, W2, b2c):
    blk = 2048
    grid = (BATCH // blk,)
    return pl.pallas_call(
        _mlp_body,
        grid=grid,
        in_specs=[
            pl.BlockSpec((blk, EMB), lambda i: (i, 0)),
            pl.BlockSpec((CAT_EMB, blk), lambda i: (0, i)),
            pl.BlockSpec((EMB, EMB), lambda i: (0, 0)),
            pl.BlockSpec((EMB, CAT_EMB), lambda i: (0, 0)),
            pl.BlockSpec((EMB, 1), lambda i: (0, 0)),
            pl.BlockSpec((EMB, EMB), lambda i: (0, 0)),
            pl.BlockSpec((EMB, 1), lambda i: (0, 0)),
        ],
        out_specs=pl.BlockSpec((EMB, blk), lambda i: (0, i)),
        out_shape=jax.ShapeDtypeStruct((EMB, BATCH), jnp.float32),
        compiler_params=pltpu.CompilerParams(
            dimension_semantics=("arbitrary",),
        ),
    )(ie, ceT, W1a, W1b, b1c, W2, b2c)


@jax.jit
def kernel(item_ids, cat_ids, item_table, cat_table, W1, b1, W2, b2):
    ie, ceT = _sc_gather(item_table.T, item_ids, cat_ids, cat_table.T)
    W1a = W1[:, :EMB]
    W1b = W1[:, EMB:]
    outT = _tc_mlp(ie, ceT, W1a, W1b, b1.reshape(EMB, 1), W2,
                   b2.reshape(EMB, 1))
    return outT.T


# speedup vs baseline: 1.0127x; 1.0127x over previous
import functools

import jax
import jax.numpy as jnp
from jax import lax
from jax.experimental import pallas as pl
from jax.experimental.pallas import tpu as pltpu
from jax.experimental.pallas import tpu_sc as plsc

N_ITEMS = 1000000
N_CATS = 1000
EMB = 64
CAT_EMB = 16
BATCH = 16384

NC = 2
NS = 16
NW = NC * NS
B_PER_W = BATCH // NW
LANES = 16
NBUF = 8
HALF = B_PER_W // 2


def _sc_gather(item_tblT, item_ids, cat_ids, cat_tblT):
    @functools.partial(
        pl.kernel,
        out_type=(
            jax.ShapeDtypeStruct((BATCH, EMB), jnp.float32),
            jax.ShapeDtypeStruct((CAT_EMB, BATCH), jnp.float32),
        ),
        mesh=plsc.VectorSubcoreMesh(core_axis_name="c", subcore_axis_name="s"),
        scratch_types=[
            pltpu.VMEM((B_PER_W,), jnp.int32),
            pltpu.VMEM((B_PER_W,), jnp.int32),
            pltpu.VMEM((NBUF, EMB, 128), jnp.float32),
            pltpu.VMEM((HALF, EMB), jnp.float32),
            pltpu.VMEM((CAT_EMB, N_CATS), jnp.float32),
            pltpu.VMEM((CAT_EMB, B_PER_W), jnp.float32),
        ] + [pltpu.SemaphoreType.DMA] * NBUF,
        compiler_params=pltpu.CompilerParams(needs_layout_passes=False),
    )
    def k(tblT_hbm, ids_hbm, cids_hbm, ctblT_hbm, ie_hbm, ceT_hbm,
          idx_v, cidx_v, stg_v, rows_v, cat_v, cdst_v, *sems):
        wid = lax.axis_index("s") * NC + lax.axis_index("c")
        base = wid * B_PER_W
        pltpu.sync_copy(ids_hbm.at[pl.ds(base, B_PER_W)], idx_v)
        iota = lax.iota(jnp.int32, LANES)

        def scalar_id(j):
            chunk = idx_v[pl.ds((j >> 4) << 4, LANES)]
            sel = jnp.where(iota == (j & 15), chunk, 0)
            return jnp.sum(sel)

        def fire(j, slot):
            tc = scalar_id(j) >> 7
            col = pl.multiple_of(tc * 128, 128)
            pltpu.async_copy(
                tblT_hbm.at[pl.ds(0, 32), pl.ds(col, 128)],
                stg_v.at[slot, pl.ds(0, 32)], sems[slot]
            )
            pltpu.async_copy(
                tblT_hbm.at[pl.ds(32, 32), pl.ds(col, 128)],
                stg_v.at[slot, pl.ds(32, 32)], sems[slot]
            )

        def extract(j, slot):
            pltpu.make_async_copy(
                tblT_hbm.at[:, pl.ds(0, 128)], stg_v.at[slot], sems[slot]
            ).wait()
            lane = jnp.full((LANES,), scalar_id(j) & 127, dtype=jnp.int32)
            for q in range(EMB // LANES):
                vals = plsc.load_gather(
                    stg_v.at[slot], [iota + (q * LANES), lane]
                )
                rows_v[j & (HALF - 1), pl.ds(q * LANES, LANES)] = vals

        for b in range(NBUF):
            fire(b, b)

        pltpu.sync_copy(cids_hbm.at[pl.ds(base, B_PER_W)], cidx_v)
        pltpu.sync_copy(ctblT_hbm, cat_v)

        @pl.loop(0, CAT_EMB)
        def _(f):
            fvec = jnp.full((LANES,), f, dtype=jnp.int32)

            @pl.loop(0, B_PER_W // LANES)
            def _(c):
                cvec = cidx_v[pl.ds(c * LANES, LANES)]
                vals = plsc.load_gather(cat_v, [fvec, cvec])
                cdst_v[f, pl.ds(c * LANES, LANES)] = vals

        pltpu.sync_copy(cdst_v, ceT_hbm.at[:, pl.ds(base, B_PER_W)])

        @pl.loop(0, B_PER_W // NBUF - 1)
        def _(g):
            for b in range(NBUF):
                j = g * NBUF + b
                extract(j, b)
                fire(j + NBUF, b)

            @pl.when(g == HALF // NBUF - 1)
            def _():
                pltpu.sync_copy(rows_v, ie_hbm.at[pl.ds(base, HALF)])

        for b in range(NBUF):
            extract(B_PER_W - NBUF + b, b)

        pltpu.sync_copy(rows_v, ie_hbm.at[pl.ds(base + HALF, HALF)])

    return k(item_tblT, item_ids, cat_ids, cat_tblT)


def _mlp_body(ie_ref, ceT_ref, w1a_ref, w1b_ref, b1_ref, w2_ref, b2_ref,
              out_ref):
    ieT = lax.dot_general(
        w1a_ref[...], ie_ref[...],
        dimension_numbers=(((1,), (1,)), ((), ())),
        preferred_element_type=jnp.float32,
    )
    h = ieT + jnp.dot(w1b_ref[...], ceT_ref[...],
                      preferred_element_type=jnp.float32)
    h = jnp.maximum(h + b1_ref[...], 0.0)
    out_ref[...] = (
        jnp.dot(w2_ref[...], h, preferred_element_type=jnp.float32)
        + b2_ref[...]
    )


def _tc_mlp(ie, ceT, W1a, W1b, b1c, W2, b2c):
    blk = 2048
    grid = (BATCH // blk,)
    return pl.pallas_call(
        _mlp_body,
        grid=grid,
        in_specs=[
            pl.BlockSpec((blk, EMB), lambda i: (i, 0)),
            pl.BlockSpec((CAT_EMB, blk), lambda i: (0, i)),
            pl.BlockSpec((EMB, EMB), lambda i: (0, 0)),
            pl.BlockSpec((EMB, CAT_EMB), lambda i: (0, 0)),
            pl.BlockSpec((EMB, 1), lambda i: (0, 0)),
            pl.BlockSpec((EMB, EMB), lambda i: (0, 0)),
            pl.BlockSpec((EMB, 1), lambda i: (0, 0)),
        ],
        out_specs=pl.BlockSpec((EMB, blk), lambda i: (0, i)),
        out_shape=jax.ShapeDtypeStruct((EMB, BATCH), jnp.float32),
        compiler_params=pltpu.CompilerParams(
            dimension_semantics=("arbitrary",),
        ),
    )(ie, ceT, W1a, W1b, b1c, W2, b2c)


@jax.jit
def kernel(item_ids, cat_ids, item_table, cat_table, W1, b1, W2, b2):
    ie, ceT = _sc_gather(item_table.T, item_ids, cat_ids, cat_table.T)
    W1a = W1[:, :EMB]
    W1b = W1[:, EMB:]
    outT = _tc_mlp(ie, ceT, W1a, W1b, b1.reshape(EMB, 1), W2,
                   b2.reshape(EMB, 1))
    return outT.T


# speedup vs baseline: 1.0277x; 1.0149x over previous
import functools

import jax
import jax.numpy as jnp
from jax import lax
from jax.experimental import pallas as pl
from jax.experimental.pallas import tpu as pltpu
from jax.experimental.pallas import tpu_sc as plsc

N_ITEMS = 1000000
N_CATS = 1000
EMB = 64
CAT_EMB = 16
BATCH = 16384

NC = 2
NS = 16
NW = NC * NS
B_PER_W = BATCH // NW
LANES = 16
NBUF = 8
HALF = B_PER_W // 2


def _sc_gather(item_tblT, item_ids, cat_ids, cat_tblT):
    @functools.partial(
        pl.kernel,
        out_type=(
            jax.ShapeDtypeStruct((BATCH, EMB), jnp.float32),
            jax.ShapeDtypeStruct((CAT_EMB, BATCH), jnp.float32),
        ),
        mesh=plsc.VectorSubcoreMesh(core_axis_name="c", subcore_axis_name="s"),
        scratch_types=[
            pltpu.VMEM((B_PER_W,), jnp.int32),
            pltpu.VMEM((B_PER_W,), jnp.int32),
            pltpu.VMEM((NBUF, EMB, 128), jnp.float32),
            pltpu.VMEM((HALF, EMB), jnp.float32),
            pltpu.VMEM((CAT_EMB, N_CATS), jnp.float32),
            pltpu.VMEM((CAT_EMB, B_PER_W), jnp.float32),
        ] + [pltpu.SemaphoreType.DMA] * NBUF,
        compiler_params=pltpu.CompilerParams(needs_layout_passes=False),
    )
    def k(tblT_hbm, ids_hbm, cids_hbm, ctblT_hbm, ie_hbm, ceT_hbm,
          idx_v, cidx_v, stg_v, rows_v, cat_v, cdst_v, *sems):
        wid = lax.axis_index("s") * NC + lax.axis_index("c")
        base = wid * B_PER_W
        pltpu.sync_copy(ids_hbm.at[pl.ds(base, B_PER_W)], idx_v)
        iota = lax.iota(jnp.int32, LANES)

        def scalar_id(j):
            chunk = idx_v[pl.ds((j >> 4) << 4, LANES)]
            sel = jnp.where(iota == (j & 15), chunk, 0)
            return jnp.sum(sel)

        def fire(j, slot):
            tc = scalar_id(j) >> 7
            col = pl.multiple_of(tc * 128, 128)
            pltpu.async_copy(
                tblT_hbm.at[:, pl.ds(col, 128)], stg_v.at[slot], sems[slot]
            )

        def extract(j, slot):
            pltpu.make_async_copy(
                tblT_hbm.at[:, pl.ds(0, 128)], stg_v.at[slot], sems[slot]
            ).wait()
            lane = jnp.full((LANES,), scalar_id(j) & 127, dtype=jnp.int32)
            for q in range(EMB // LANES):
                vals = plsc.load_gather(
                    stg_v.at[slot], [iota + (q * LANES), lane]
                )
                rows_v[j & (HALF - 1), pl.ds(q * LANES, LANES)] = vals

        for b in range(NBUF):
            fire(b, b)

        pltpu.sync_copy(cids_hbm.at[pl.ds(base, B_PER_W)], cidx_v)
        pltpu.sync_copy(ctblT_hbm, cat_v)

        @pl.loop(0, CAT_EMB)
        def _(f):
            fvec = jnp.full((LANES,), f, dtype=jnp.int32)

            @pl.loop(0, B_PER_W // LANES)
            def _(c):
                cvec = cidx_v[pl.ds(c * LANES, LANES)]
                vals = plsc.load_gather(cat_v, [fvec, cvec])
                cdst_v[f, pl.ds(c * LANES, LANES)] = vals

        pltpu.sync_copy(cdst_v, ceT_hbm.at[:, pl.ds(base, B_PER_W)])

        @pl.loop(0, B_PER_W // NBUF - 1)
        def _(g):
            for b in range(NBUF):
                j = g * NBUF + b
                extract(j, b)
                fire(j + NBUF, b)

            @pl.when(g == HALF // NBUF - 1)
            def _():
                pltpu.sync_copy(rows_v, ie_hbm.at[pl.ds(base, HALF)])

        for b in range(NBUF):
            extract(B_PER_W - NBUF + b, b)

        pltpu.sync_copy(rows_v, ie_hbm.at[pl.ds(base + HALF, HALF)])

    return k(item_tblT, item_ids, cat_ids, cat_tblT)


def _mlp_body(ie_ref, ceT_ref, w1a_ref, w1b_ref, b1_ref, w2_ref, b2_ref,
              out_ref):
    ieT = lax.dot_general(
        w1a_ref[...], ie_ref[...],
        dimension_numbers=(((1,), (1,)), ((), ())),
        preferred_element_type=jnp.float32,
    )
    h = ieT + jnp.dot(w1b_ref[...], ceT_ref[...],
                      preferred_element_type=jnp.float32)
    h = jnp.maximum(h + b1_ref[...], 0.0)
    out_ref[...] = (
        jnp.dot(w2_ref[...], h, preferred_element_type=jnp.float32)
        + b2_ref[...]
    )


def _tc_mlp(ie, ceT, W1a, W1b, b1c, W2, b2c):
    blk = 2048
    grid = (BATCH // blk,)
    return pl.pallas_call(
        _mlp_body,
        grid=grid,
        in_specs=[
            pl.BlockSpec((blk, EMB), lambda i: (i, 0)),
            pl.BlockSpec((CAT_EMB, blk), lambda i: (0, i)),
            pl.BlockSpec((EMB, EMB), lambda i: (0, 0)),
            pl.BlockSpec((EMB, CAT_EMB), lambda i: (0, 0)),
            pl.BlockSpec((EMB, 1), lambda i: (0, 0)),
            pl.BlockSpec((EMB, EMB), lambda i: (0, 0)),
            pl.BlockSpec((EMB, 1), lambda i: (0, 0)),
        ],
        out_specs=pl.BlockSpec((EMB, blk), lambda i: (0, i)),
        out_shape=jax.ShapeDtypeStruct((EMB, BATCH), jnp.float32),
        compiler_params=pltpu.CompilerParams(
            dimension_semantics=("arbitrary",),
        ),
    )(ie, ceT, W1a, W1b, b1c, W2, b2c)


@jax.jit
def kernel(item_ids, cat_ids, item_table, cat_table, W1, b1, W2, b2):
    ie, ceT = _sc_gather(item_table.T, item_ids, cat_ids, cat_table.T)
    W1a = W1[:, :EMB]
    W1b = W1[:, EMB:]
    outT = _tc_mlp(ie, ceT, W1a, W1b, b1.reshape(EMB, 1), W2,
                   b2.reshape(EMB, 1))
    return outT.T
